# Initial kernel scaffold; baseline (speedup 1.0000x reference)
import functools

import jax
import jax.numpy as jnp
from jax import lax
from jax.experimental import pallas as pl
from jax.experimental.pallas import tpu as pltpu
from jax.experimental.pallas import tpu_sc as plsc

N = 10000
E = 320000
NFEAT = 128
NHID = 64
NCLASS = 16
H2 = NHID * 2
H3 = NHID * 3

NTILES = 32
EPT = E // NTILES
CH = 100
NCH = EPT // CH
RPS = N // 16

_f32 = jnp.float32


def _fold_body(w_org, b_org, wg1, wg2, wl, w3, b3, bg2, bl,
               w1_o, c1_o, w2l_o, w3l_o, c3_o):
    w_org_v = w_org[...]
    b_org_v = b_org[...]
    wl_v = wl[...]
    w3_v = w3[...]
    w1_o[...] = jnp.dot(w_org_v, wg1[...], preferred_element_type=_f32)
    c1_o[...] = jnp.dot(b_org_v, wg1[...], preferred_element_type=_f32)
    w2l_o[...] = jnp.dot(wg2[...], wl_v, preferred_element_type=_f32)
    w3l_o[...] = jnp.dot(jnp.dot(w_org_v, w3_v, preferred_element_type=_f32),
                         wl_v, preferred_element_type=_f32)
    c3_o[...] = (jnp.dot(jnp.dot(b_org_v, w3_v, preferred_element_type=_f32)
                         + b3[...] + 0.5 * bg2[...],
                         wl_v, preferred_element_type=_f32) + bl[...])


_fold = pl.pallas_call(
    _fold_body,
    out_shape=(
        jax.ShapeDtypeStruct((H2, H3), _f32),
        jax.ShapeDtypeStruct((1, H3), _f32),
        jax.ShapeDtypeStruct((H3, NCLASS), _f32),
        jax.ShapeDtypeStruct((H2, NCLASS), _f32),
        jax.ShapeDtypeStruct((1, NCLASS), _f32),
    ),
)


def _lin_a_body(x_ref, w1_ref, c1_ref, w3l_ref, c3_ref, s1_ref, r3_ref):
    xb = x_ref[...]
    s1_ref[...] = jnp.dot(xb, w1_ref[...], preferred_element_type=_f32) + c1_ref[...]
    r3_ref[...] = jnp.dot(xb, w3l_ref[...], preferred_element_type=_f32) + c3_ref[...]


_BR = 1000

_lin_a = pl.pallas_call(
    _lin_a_body,
    grid=(N // _BR,),
    in_specs=[
        pl.BlockSpec((_BR, NFEAT), lambda i: (i, 0)),
        pl.BlockSpec((H2, H3), lambda i: (0, 0)),
        pl.BlockSpec((1, H3), lambda i: (0, 0)),
        pl.BlockSpec((H2, NCLASS), lambda i: (0, 0)),
        pl.BlockSpec((1, NCLASS), lambda i: (0, 0)),
    ],
    out_specs=(
        pl.BlockSpec((_BR, H3), lambda i: (i, 0)),
        pl.BlockSpec((_BR, NCLASS), lambda i: (i, 0)),
    ),
    out_shape=(
        jax.ShapeDtypeStruct((N, H3), _f32),
        jax.ShapeDtypeStruct((N, NCLASS), _f32),
    ),
)


def _lin_b_body(p1_ref, bg1_ref, w2l_ref, s2_ref):
    a = p1_ref[0] + p1_ref[1] + bg1_ref[...]
    g1 = jnp.maximum(a, 0.0)
    s2_ref[...] = jnp.dot(g1, w2l_ref[...], preferred_element_type=_f32)


_lin_b = pl.pallas_call(
    _lin_b_body,
    grid=(N // _BR,),
    in_specs=[
        pl.BlockSpec((2, _BR, H3), lambda i: (0, i, 0)),
        pl.BlockSpec((1, H3), lambda i: (0, 0)),
        pl.BlockSpec((H3, NCLASS), lambda i: (0, 0)),
    ],
    out_specs=pl.BlockSpec((_BR, NCLASS), lambda i: (i, 0)),
    out_shape=jax.ShapeDtypeStruct((N, NCLASS), _f32),
)


def _final_body(r3_ref, p2_ref, out_ref):
    t = r3_ref[...] + 0.5 * (p2_ref[0] + p2_ref[1])
    mn = jnp.min(t)
    mx = jnp.max(t)
    out_ref[...] = 2.0 * (t - mn) / (mx - mn) - 1.0


_final = pl.pallas_call(
    _final_body,
    out_shape=jax.ShapeDtypeStruct((N * NCLASS // 128, 128), _f32),
)


def _make_spmm(nf):
    nvec = nf // 16
    mesh = plsc.VectorSubcoreMesh(core_axis_name="c", subcore_axis_name="s")

    def body(table, srcg, dstg, wg, zrows, out,
             idx_all, dst_all, w_all, rows_v, acc_sh, sem):
        c = lax.axis_index("c")
        s = lax.axis_index("s")
        tid = s * 2 + c
        pltpu.sync_copy(zrows, acc_sh.at[pl.ds(s * RPS, RPS)])
        pltpu.sync_copy(srcg.at[tid], idx_all)
        pltpu.sync_copy(dstg.at[tid], dst_all)
        pltpu.sync_copy(wg.at[tid], w_all)
        plsc.subcore_barrier()

        def chunk(g, _):
            pltpu.async_copy(table.at[idx_all.at[g]], rows_v, sem).wait()

            def row(r, _):
                wspl = plsc.load_gather(
                    w_all,
                    [jnp.full((16,), g, jnp.int32), jnp.full((16,), r, jnp.int32)])
                for j in range(nvec):
                    rows_v[r, pl.ds(j * 16, 16)] = rows_v[r, pl.ds(j * 16, 16)] * wspl
                return 0

            lax.fori_loop(0, CH, row, 0, unroll=2)
            pltpu.sync_copy(rows_v, acc_sh.at[dst_all.at[g]], add=True)
            return 0

        lax.fori_loop(0, NCH, chunk, 0)
        plsc.subcore_barrier()
        pltpu.sync_copy(acc_sh.at[pl.ds(s * RPS, RPS)],
                        out.at[c, pl.ds(s * RPS, RPS)])

    return pl.kernel(
        body,
        out_type=jax.ShapeDtypeStruct((2, N, nf), _f32),
        mesh=mesh,
        scratch_types=[
            pltpu.VMEM((NCH, CH), jnp.int32),
            pltpu.VMEM((NCH, CH), jnp.int32),
            pltpu.VMEM((NCH, CH), _f32),
            pltpu.VMEM((CH, nf), _f32),
            pltpu.VMEM_SHARED((N, nf), _f32),
            pltpu.SemaphoreType.DMA,
        ],
    )


_spmm_h3 = _make_spmm(H3)
_spmm_cls = _make_spmm(NCLASS)


def kernel(x, edge_index, edge_weight, W_org, b_org, Wg1, bg1, Wg2, bg2,
           W3, b3, Wl, bl):
    dst = edge_index[0].astype(jnp.int32).reshape(NTILES, NCH, CH)
    src = edge_index[1].astype(jnp.int32).reshape(NTILES, NCH, CH)
    w = edge_weight.reshape(NTILES, NCH, CH)

    w1, c1, w2l, w3l, c3 = _fold(
        W_org, b_org.reshape(1, -1), Wg1, Wg2, Wl, W3,
        b3.reshape(1, -1), bg2.reshape(1, -1), bl.reshape(1, -1))

    s1, r3 = _lin_a(x, w1, c1, w3l, c3)

    z_h3 = jnp.zeros((RPS, H3), _f32)
    p1 = _spmm_h3(s1, src, dst, w, z_h3)

    s2 = _lin_b(p1, bg1.reshape(1, -1), w2l)

    z_cls = jnp.zeros((RPS, NCLASS), _f32)
    p2 = _spmm_cls(s2, src, dst, w, z_cls)

    out = _final(r3.reshape(N * NCLASS // 128, 128),
                 p2.reshape(2, N * NCLASS // 128, 128))
    return out.reshape(N, NCLASS)


# speedup vs baseline: 6.1422x; 6.1422x over previous
import jax
import jax.numpy as jnp
from jax import lax
from jax.experimental import pallas as pl
from jax.experimental.pallas import tpu as pltpu
from jax.experimental.pallas import tpu_sc as plsc

N = 10000
E = 320000
NFEAT = 128
NHID = 64
NCLASS = 16
H2 = NHID * 2
H3 = NHID * 3
NFP = 144

NTILES = 32
EPT = E // NTILES
CH = 100
NCH = EPT // CH
NPAD = 10240
RPS = NPAD // 16

_f32 = jnp.float32


def _fold_body(w_org, b_org, wg1, wg2, wl, w3, b3, bg2, bl,
               w1_o, c1_o, w2l_o, w3l_o, c3_o):
    w_org_v = w_org[...]
    b_org_v = b_org[...]
    wl_v = wl[...]
    w3_v = w3[...]
    w1_o[...] = jnp.dot(w_org_v, wg1[...], preferred_element_type=_f32)
    c1_o[...] = jnp.dot(b_org_v, wg1[...], preferred_element_type=_f32)
    w2l_o[...] = jnp.dot(wg2[...], wl_v, preferred_element_type=_f32)
    w3l_o[...] = jnp.dot(jnp.dot(w_org_v, w3_v, preferred_element_type=_f32),
                         wl_v, preferred_element_type=_f32)
    c3_o[...] = (jnp.dot(jnp.dot(b_org_v, w3_v, preferred_element_type=_f32)
                         + b3[...] + 0.5 * bg2[...],
                         wl_v, preferred_element_type=_f32) + bl[...])


_fold = pl.pallas_call(
    _fold_body,
    out_shape=(
        jax.ShapeDtypeStruct((H2, H3), _f32),
        jax.ShapeDtypeStruct((1, H3), _f32),
        jax.ShapeDtypeStruct((H3, NCLASS), _f32),
        jax.ShapeDtypeStruct((H2, NCLASS), _f32),
        jax.ShapeDtypeStruct((1, NCLASS), _f32),
    ),
)


_BR = 1000


def _lin_b_body(p1_ref, x_ref, bg1_ref, w1_ref, c1_ref, w2l_ref, w3l_ref,
                c3_ref, s2_ref, r3_ref):
    ax = p1_ref[0, :, :H2] + p1_ref[1, :, :H2]
    deg = p1_ref[0, :, H2:H2 + 1] + p1_ref[1, :, H2:H2 + 1]
    a = (jnp.dot(ax, w1_ref[...], preferred_element_type=_f32)
         + deg * c1_ref[...] + bg1_ref[...])
    g1 = jnp.maximum(a, 0.0)
    s2_ref[...] = jnp.dot(g1, w2l_ref[...], preferred_element_type=_f32)
    r3_ref[...] = (jnp.dot(x_ref[...], w3l_ref[...],
                           preferred_element_type=_f32) + c3_ref[...])


_lin_b = pl.pallas_call(
    _lin_b_body,
    grid=(N // _BR,),
    in_specs=[
        pl.BlockSpec((2, _BR, NFP), lambda i: (0, i, 0)),
        pl.BlockSpec((_BR, NFEAT), lambda i: (i, 0)),
        pl.BlockSpec((1, H3), lambda i: (0, 0)),
        pl.BlockSpec((H2, H3), lambda i: (0, 0)),
        pl.BlockSpec((1, H3), lambda i: (0, 0)),
        pl.BlockSpec((H3, NCLASS), lambda i: (0, 0)),
        pl.BlockSpec((H2, NCLASS), lambda i: (0, 0)),
        pl.BlockSpec((1, NCLASS), lambda i: (0, 0)),
    ],
    out_specs=(
        pl.BlockSpec((_BR, NCLASS), lambda i: (i, 0)),
        pl.BlockSpec((_BR, NCLASS), lambda i: (i, 0)),
    ),
    out_shape=(
        jax.ShapeDtypeStruct((N, NCLASS), _f32),
        jax.ShapeDtypeStruct((N, NCLASS), _f32),
    ),
)


def _final_body(r3_ref, p2_ref, out_ref):
    t = r3_ref[...] + 0.5 * (p2_ref[0] + p2_ref[1])
    mn = jnp.min(t)
    mx = jnp.max(t)
    out_ref[...] = 2.0 * (t - mn) / (mx - mn) - 1.0


_NR = N * NCLASS // 128

_final = pl.pallas_call(
    _final_body,
    out_shape=jax.ShapeDtypeStruct((_NR, 128), _f32),
)


def _make_spmm(nf):
    nvec = nf // 16
    mesh = plsc.VectorSubcoreMesh(core_axis_name="c", subcore_axis_name="s")

    def body(table, srcg, dstg, wg, zrows, out,
             idx_v, dst_all, w_all, rows_v, acc_sh, sem):
        c = lax.axis_index("c")
        s = lax.axis_index("s")
        tid = s * 2 + c
        pltpu.sync_copy(zrows, acc_sh.at[pl.ds(s * RPS, RPS)])
        pltpu.sync_copy(dstg.at[tid], dst_all)
        pltpu.sync_copy(wg.at[tid], w_all)
        plsc.subcore_barrier()

        def chunk(g, _):
            pltpu.sync_copy(srcg.at[tid, g], idx_v)
            pltpu.async_copy(table.at[idx_v], rows_v, sem).wait()

            def row(r, _):
                wspl = plsc.load_gather(
                    w_all, [jnp.full((16,), g * CH + r, jnp.int32)])
                for j in range(nvec):
                    rows_v[r, pl.ds(j * 16, 16)] = rows_v[r, pl.ds(j * 16, 16)] * wspl
                return 0

            lax.fori_loop(0, CH, row, 0, unroll=2)
            pltpu.sync_copy(rows_v, acc_sh.at[dst_all.at[g]], add=True)
            return 0

        lax.fori_loop(0, NCH, chunk, 0)
        plsc.subcore_barrier()
        pltpu.sync_copy(acc_sh.at[pl.ds(s * RPS, RPS)],
                        out.at[c, pl.ds(s * RPS, RPS)])

    return pl.kernel(
        body,
        out_type=jax.ShapeDtypeStruct((2, NPAD, nf), _f32),
        mesh=mesh,
        compiler_params=pltpu.CompilerParams(needs_layout_passes=False,
                                             use_tc_tiling_on_sc=False),
        scratch_types=[
            pltpu.VMEM((CH,), jnp.int32),
            pltpu.VMEM((NCH, CH), jnp.int32),
            pltpu.VMEM((EPT,), _f32),
            pltpu.VMEM((CH, nf), _f32),
            pltpu.VMEM_SHARED((NPAD, nf), _f32),
            pltpu.SemaphoreType.DMA,
        ],
    )


_spmm_x = _make_spmm(NFP)
_spmm_cls = _make_spmm(NCLASS)


def kernel(x, edge_index, edge_weight, W_org, b_org, Wg1, bg1, Wg2, bg2,
           W3, b3, Wl, bl):
    dst = edge_index[0].astype(jnp.int32).reshape(NTILES, NCH, CH)
    src = edge_index[1].astype(jnp.int32).reshape(NTILES, NCH, CH)
    w = edge_weight.reshape(NTILES, EPT)

    xp = jnp.zeros((N, NFP), _f32)
    xp = xp.at[:, :NFEAT].set(x)
    xp = xp.at[:, H2].set(1.0)

    w1, c1, w2l, w3l, c3 = _fold(
        W_org, b_org.reshape(1, -1), Wg1, Wg2, Wl, W3,
        b3.reshape(1, -1), bg2.reshape(1, -1), bl.reshape(1, -1))

    z_x = jnp.zeros((RPS, NFP), _f32)
    p1 = _spmm_x(xp, src, dst, w, z_x)

    s2, r3 = _lin_b(p1, x, bg1.reshape(1, -1), w1, c1, w2l, w3l,
                    c3.reshape(1, -1))

    z_cls = jnp.zeros((RPS, NCLASS), _f32)
    p2 = _spmm_cls(s2, src, dst, w, z_cls)

    out = _final(r3.reshape(_NR, 128), p2[:, :N, :].reshape(2, _NR, 128))
    return out.reshape(N, NCLASS)


# speedup vs baseline: 7.2325x; 1.1775x over previous
import jax
import jax.numpy as jnp
from jax import lax
from jax.experimental import pallas as pl
from jax.experimental.pallas import tpu as pltpu
from jax.experimental.pallas import tpu_sc as plsc

N = 10000
E = 320000
NFEAT = 128
NHID = 64
NCLASS = 16
H2 = NHID * 2
H3 = NHID * 3
NFP = 144

NTILES = 32
EPT = E // NTILES
CH = 100
NCH = EPT // CH
NPAD = 10112
RPS = NPAD // 16

_f32 = jnp.float32


def _fold_body(w_org, b_org, wg1, wg2, wl, w3, b3, bg2, bl,
               w1_o, c1_o, w2l_o, w3l_o, c3_o):
    w_org_v = w_org[...]
    b_org_v = b_org[...]
    wl_v = wl[...]
    w3_v = w3[...]
    w1_o[...] = jnp.dot(w_org_v, wg1[...], preferred_element_type=_f32)
    c1_o[...] = jnp.dot(b_org_v, wg1[...], preferred_element_type=_f32)
    w2l_o[...] = jnp.dot(wg2[...], wl_v, preferred_element_type=_f32)
    w3l_o[...] = jnp.dot(jnp.dot(w_org_v, w3_v, preferred_element_type=_f32),
                         wl_v, preferred_element_type=_f32)
    c3_o[...] = (jnp.dot(jnp.dot(b_org_v, w3_v, preferred_element_type=_f32)
                         + b3[...] + 0.5 * bg2[...],
                         wl_v, preferred_element_type=_f32) + bl[...])


_fold = pl.pallas_call(
    _fold_body,
    out_shape=(
        jax.ShapeDtypeStruct((H2, H3), _f32),
        jax.ShapeDtypeStruct((1, H3), _f32),
        jax.ShapeDtypeStruct((H3, NCLASS), _f32),
        jax.ShapeDtypeStruct((H2, NCLASS), _f32),
        jax.ShapeDtypeStruct((1, NCLASS), _f32),
    ),
)


_BR = 1000


def _lin_b_body(p1_ref, x_ref, bg1_ref, w1_ref, c1_ref, w2l_ref, w3l_ref,
                c3_ref, s2_ref, r3_ref):
    ax = p1_ref[0, :, :H2] + p1_ref[1, :, :H2]
    deg = p1_ref[0, :, H2:H2 + 1] + p1_ref[1, :, H2:H2 + 1]
    a = (jnp.dot(ax, w1_ref[...], preferred_element_type=_f32)
         + deg * c1_ref[...] + bg1_ref[...])
    g1 = jnp.maximum(a, 0.0)
    s2_ref[...] = jnp.dot(g1, w2l_ref[...], preferred_element_type=_f32)
    r3_ref[...] = (jnp.dot(x_ref[...], w3l_ref[...],
                           preferred_element_type=_f32) + c3_ref[...])


_lin_b = pl.pallas_call(
    _lin_b_body,
    grid=(N // _BR,),
    in_specs=[
        pl.BlockSpec((2, _BR, NFP), lambda i: (0, i, 0)),
        pl.BlockSpec((_BR, NFEAT), lambda i: (i, 0)),
        pl.BlockSpec((1, H3), lambda i: (0, 0)),
        pl.BlockSpec((H2, H3), lambda i: (0, 0)),
        pl.BlockSpec((1, H3), lambda i: (0, 0)),
        pl.BlockSpec((H3, NCLASS), lambda i: (0, 0)),
        pl.BlockSpec((H2, NCLASS), lambda i: (0, 0)),
        pl.BlockSpec((1, NCLASS), lambda i: (0, 0)),
    ],
    out_specs=(
        pl.BlockSpec((_BR, NCLASS), lambda i: (i, 0)),
        pl.BlockSpec((_BR, NCLASS), lambda i: (i, 0)),
    ),
    out_shape=(
        jax.ShapeDtypeStruct((N, NCLASS), _f32),
        jax.ShapeDtypeStruct((N, NCLASS), _f32),
    ),
)


def _final_body(r3_ref, p2_ref, out_ref):
    t = r3_ref[...] + 0.5 * (p2_ref[0] + p2_ref[1])
    mn = jnp.min(t)
    mx = jnp.max(t)
    out_ref[...] = 2.0 * (t - mn) / (mx - mn) - 1.0


_NR = N * NCLASS // 128

_final = pl.pallas_call(
    _final_body,
    out_shape=jax.ShapeDtypeStruct((_NR, 128), _f32),
)


def _make_spmm(nf, mul_unroll):
    nvec = nf // 16
    mesh = plsc.VectorSubcoreMesh(core_axis_name="c", subcore_axis_name="s")

    def body(table, srcg, dstg, wexpg, zrows, out,
             idx0, idx1, dst0, dst1, wx0, wx1, rows0, rows1, acc_sh,
             sf0, sf1, sd0, sd1, sg0, sg1, ss0, ss1):
        c = lax.axis_index("c")
        s = lax.axis_index("s")
        tid = s * 2 + c
        idx_b = (idx0, idx1)
        dst_b = (dst0, dst1)
        wx_b = (wx0, wx1)
        rows_b = (rows0, rows1)
        sf = (sf0, sf1)
        sd = (sd0, sd1)
        sg = (sg0, sg1)
        ss = (ss0, ss1)

        pltpu.sync_copy(zrows, acc_sh.at[pl.ds(s * RPS, RPS)])
        plsc.subcore_barrier()

        def mul(b):
            rows_v = rows_b[b]
            wx_v = wx_b[b]

            def row(r, _):
                wspl = wx_v[r, :]
                for j in range(nvec):
                    rows_v[r, pl.ds(j * 16, 16)] = (
                        rows_v[r, pl.ds(j * 16, 16)] * wspl)
                return 0

            lax.fori_loop(0, CH, row, 0, unroll=mul_unroll)

        pltpu.async_copy(srcg.at[tid, 0], idx_b[0], sf[0])
        pltpu.async_copy(wexpg.at[tid, 0], wx_b[0], sf[0])
        pltpu.async_copy(srcg.at[tid, 1], idx_b[1], sf[1])
        pltpu.async_copy(wexpg.at[tid, 1], wx_b[1], sf[1])
        pltpu.async_copy(dstg.at[tid, 0], dst_b[0], sd[0])
        pltpu.make_async_copy(srcg.at[tid, 0], idx_b[0], sf[0]).wait()
        pltpu.make_async_copy(wexpg.at[tid, 0], wx_b[0], sf[0]).wait()
        pltpu.async_copy(table.at[idx_b[0]], rows_b[0], sg[0])

        def kstep(k, _):
            for b in (0, 1):
                g = 2 * k + b
                o = 1 - b

                @pl.when(g + 1 < NCH)
                def _():
                    pltpu.make_async_copy(
                        srcg.at[tid, g + 1], idx_b[o], sf[o]).wait()
                    pltpu.make_async_copy(
                        wexpg.at[tid, g + 1], wx_b[o], sf[o]).wait()

                @pl.when(g >= 1)
                def _():
                    pltpu.make_async_copy(
                        rows_b[o], acc_sh.at[dst_b[o]], ss[o]).wait()

                @pl.when(g + 1 < NCH)
                def _():
                    pltpu.async_copy(dstg.at[tid, g + 1], dst_b[o], sd[o])
                    pltpu.async_copy(table.at[idx_b[o]], rows_b[o], sg[o])

                pltpu.make_async_copy(table.at[idx_b[b]], rows_b[b], sg[b]).wait()
                mul(b)

                @pl.when(g + 2 < NCH)
                def _():
                    pltpu.async_copy(srcg.at[tid, g + 2], idx_b[b], sf[b])
                    pltpu.async_copy(wexpg.at[tid, g + 2], wx_b[b], sf[b])

                pltpu.make_async_copy(dstg.at[tid, g], dst_b[b], sd[b]).wait()

                @pl.when(g < NCH - 1)
                def _():
                    pltpu.async_copy(rows_b[b], acc_sh.at[dst_b[b]], ss[b],
                                     add=True)

                @pl.when(g == NCH - 1)
                def _():
                    pltpu.sync_copy(rows_b[b], acc_sh.at[dst_b[b]], add=True)
            return 0

        lax.fori_loop(0, NCH // 2, kstep, 0)
        plsc.subcore_barrier()
        pltpu.sync_copy(acc_sh.at[pl.ds(s * RPS, RPS)],
                        out.at[c, pl.ds(s * RPS, RPS)])

    return pl.kernel(
        body,
        out_type=jax.ShapeDtypeStruct((2, NPAD, nf), _f32),
        mesh=mesh,
        compiler_params=pltpu.CompilerParams(needs_layout_passes=False,
                                             use_tc_tiling_on_sc=False),
        scratch_types=[
            pltpu.VMEM((CH,), jnp.int32),
            pltpu.VMEM((CH,), jnp.int32),
            pltpu.VMEM((CH,), jnp.int32),
            pltpu.VMEM((CH,), jnp.int32),
            pltpu.VMEM((CH, 16), _f32),
            pltpu.VMEM((CH, 16), _f32),
            pltpu.VMEM((CH, nf), _f32),
            pltpu.VMEM((CH, nf), _f32),
            pltpu.VMEM_SHARED((NPAD, nf), _f32),
            pltpu.SemaphoreType.DMA,
            pltpu.SemaphoreType.DMA,
            pltpu.SemaphoreType.DMA,
            pltpu.SemaphoreType.DMA,
            pltpu.SemaphoreType.DMA,
            pltpu.SemaphoreType.DMA,
            pltpu.SemaphoreType.DMA,
            pltpu.SemaphoreType.DMA,
        ],
    )


_spmm_x = _make_spmm(NFP, 2)
_spmm_cls = _make_spmm(NCLASS, 8)


def kernel(x, edge_index, edge_weight, W_org, b_org, Wg1, bg1, Wg2, bg2,
           W3, b3, Wl, bl):
    dst = edge_index[0].astype(jnp.int32).reshape(NTILES, NCH, CH)
    src = edge_index[1].astype(jnp.int32).reshape(NTILES, NCH, CH)
    wexp = jnp.broadcast_to(edge_weight.reshape(E, 1),
                            (E, 16)).reshape(NTILES, NCH, CH, 16)

    xp = jnp.zeros((N, NFP), _f32)
    xp = xp.at[:, :NFEAT].set(x)
    xp = xp.at[:, H2].set(1.0)

    w1, c1, w2l, w3l, c3 = _fold(
        W_org, b_org.reshape(1, -1), Wg1, Wg2, Wl, W3,
        b3.reshape(1, -1), bg2.reshape(1, -1), bl.reshape(1, -1))

    z_x = jnp.zeros((RPS, NFP), _f32)
    p1 = _spmm_x(xp, src, dst, wexp, z_x)

    s2, r3 = _lin_b(p1, x, bg1.reshape(1, -1), w1, c1, w2l, w3l,
                    c3.reshape(1, -1))

    z_cls = jnp.zeros((RPS, NCLASS), _f32)
    p2 = _spmm_cls(s2, src, dst, wexp, z_cls)

    out = _final(r3.reshape(_NR, 128), p2[:, :N, :].reshape(2, _NR, 128))
    return out.reshape(N, NCLASS)


# speedup vs baseline: 10.1477x; 1.4031x over previous
import jax
import jax.numpy as jnp
from jax import lax
from jax.experimental import pallas as pl
from jax.experimental.pallas import tpu as pltpu
from jax.experimental.pallas import tpu_sc as plsc

N = 10000
E = 320000
NFEAT = 128
NHID = 64
NCLASS = 16
H2 = NHID * 2
H3 = NHID * 3
NFP = 144

NTILES = 32
EPT = E // NTILES
CH = 125
NCH = EPT // CH
NPAD = 10112
RPS = NPAD // 16

_f32 = jnp.float32


def _fold_body(w_org, b_org, wg1, wg2, wl, w3, b3, bg2, bl,
               w1_o, c1_o, w2l_o, w3l_o, c3_o):
    w_org_v = w_org[...]
    b_org_v = b_org[...]
    wl_v = wl[...]
    w3_v = w3[...]
    w1_o[...] = jnp.dot(w_org_v, wg1[...], preferred_element_type=_f32)
    c1_o[...] = jnp.dot(b_org_v, wg1[...], preferred_element_type=_f32)
    w2l_o[...] = jnp.dot(wg2[...], wl_v, preferred_element_type=_f32)
    w3l_o[...] = jnp.dot(jnp.dot(w_org_v, w3_v, preferred_element_type=_f32),
                         wl_v, preferred_element_type=_f32)
    c3_o[...] = (jnp.dot(jnp.dot(b_org_v, w3_v, preferred_element_type=_f32)
                         + b3[...] + 0.5 * bg2[...],
                         wl_v, preferred_element_type=_f32) + bl[...])


_fold = pl.pallas_call(
    _fold_body,
    out_shape=(
        jax.ShapeDtypeStruct((H2, H3), _f32),
        jax.ShapeDtypeStruct((1, H3), _f32),
        jax.ShapeDtypeStruct((H3, NCLASS), _f32),
        jax.ShapeDtypeStruct((H2, NCLASS), _f32),
        jax.ShapeDtypeStruct((1, NCLASS), _f32),
    ),
)


_BR = 1000


def _lin_b_body(p1_ref, x_ref, bg1_ref, w1_ref, c1_ref, w2l_ref, w3l_ref,
                c3_ref, s2_ref, r3_ref):
    ax = p1_ref[0, :, :H2] + p1_ref[1, :, :H2]
    deg = p1_ref[0, :, H2:H2 + 1] + p1_ref[1, :, H2:H2 + 1]
    a = (jnp.dot(ax, w1_ref[...], preferred_element_type=_f32)
         + deg * c1_ref[...] + bg1_ref[...])
    g1 = jnp.maximum(a, 0.0)
    s2_ref[...] = jnp.dot(g1, w2l_ref[...], preferred_element_type=_f32)
    r3_ref[...] = (jnp.dot(x_ref[...], w3l_ref[...],
                           preferred_element_type=_f32) + c3_ref[...])


_lin_b = pl.pallas_call(
    _lin_b_body,
    grid=(N // _BR,),
    in_specs=[
        pl.BlockSpec((2, _BR, NFP), lambda i: (0, i, 0)),
        pl.BlockSpec((_BR, NFEAT), lambda i: (i, 0)),
        pl.BlockSpec((1, H3), lambda i: (0, 0)),
        pl.BlockSpec((H2, H3), lambda i: (0, 0)),
        pl.BlockSpec((1, H3), lambda i: (0, 0)),
        pl.BlockSpec((H3, NCLASS), lambda i: (0, 0)),
        pl.BlockSpec((H2, NCLASS), lambda i: (0, 0)),
        pl.BlockSpec((1, NCLASS), lambda i: (0, 0)),
    ],
    out_specs=(
        pl.BlockSpec((_BR, NCLASS), lambda i: (i, 0)),
        pl.BlockSpec((_BR, NCLASS), lambda i: (i, 0)),
    ),
    out_shape=(
        jax.ShapeDtypeStruct((N, NCLASS), _f32),
        jax.ShapeDtypeStruct((N, NCLASS), _f32),
    ),
)


def _final_body(r3_ref, p2_ref, out_ref):
    t = r3_ref[...] + 0.5 * (p2_ref[0] + p2_ref[1])
    mn = jnp.min(t)
    mx = jnp.max(t)
    out_ref[...] = 2.0 * (t - mn) / (mx - mn) - 1.0


_NR = N * NCLASS // 128

_final = pl.pallas_call(
    _final_body,
    out_shape=jax.ShapeDtypeStruct((_NR, 128), _f32),
)


def _make_spmm(nf, mul_unroll):
    nvec = nf // 16
    mesh = plsc.VectorSubcoreMesh(core_axis_name="c", subcore_axis_name="s")

    def body(table, srcg, dstg, wexpg, zrows, out,
             idx0, idx1, dst0, dst1, wx0, wx1, rows0, rows1, acc_sh,
             sf0, sf1, sd0, sd1, sg0, sg1, ss0, ss1):
        c = lax.axis_index("c")
        s = lax.axis_index("s")
        tid = s * 2 + c
        idx_b = (idx0, idx1)
        dst_b = (dst0, dst1)
        wx_b = (wx0, wx1)
        rows_b = (rows0, rows1)
        sf = (sf0, sf1)
        sd = (sd0, sd1)
        sg = (sg0, sg1)
        ss = (ss0, ss1)

        pltpu.sync_copy(zrows, acc_sh.at[pl.ds(s * RPS, RPS)])
        plsc.subcore_barrier()

        def mul(b):
            rows_v = rows_b[b]
            wx_v = wx_b[b]

            def row(r, _):
                wspl = plsc.load_gather(wx_v, [jnp.full((16,), r, jnp.int32)])
                for j in range(nvec):
                    rows_v[r, pl.ds(j * 16, 16)] = (
                        rows_v[r, pl.ds(j * 16, 16)] * wspl)
                return 0

            lax.fori_loop(0, CH, row, 0, unroll=mul_unroll)

        pltpu.async_copy(srcg.at[tid, 0], idx_b[0], sf[0])
        pltpu.async_copy(wexpg.at[tid, 0], wx_b[0], sf[0])
        pltpu.async_copy(srcg.at[tid, 1], idx_b[1], sf[1])
        pltpu.async_copy(wexpg.at[tid, 1], wx_b[1], sf[1])
        pltpu.async_copy(dstg.at[tid, 0], dst_b[0], sd[0])
        pltpu.make_async_copy(srcg.at[tid, 0], idx_b[0], sf[0]).wait()
        pltpu.make_async_copy(wexpg.at[tid, 0], wx_b[0], sf[0]).wait()
        pltpu.async_copy(table.at[idx_b[0]], rows_b[0], sg[0])

        def kstep(k, _):
            for b in (0, 1):
                g = 2 * k + b
                o = 1 - b

                @pl.when(g + 1 < NCH)
                def _():
                    pltpu.make_async_copy(
                        srcg.at[tid, g + 1], idx_b[o], sf[o]).wait()
                    pltpu.make_async_copy(
                        wexpg.at[tid, g + 1], wx_b[o], sf[o]).wait()

                @pl.when(g >= 1)
                def _():
                    pltpu.make_async_copy(
                        rows_b[o], acc_sh.at[dst_b[o]], ss[o]).wait()

                @pl.when(g + 1 < NCH)
                def _():
                    pltpu.async_copy(dstg.at[tid, g + 1], dst_b[o], sd[o])
                    pltpu.async_copy(table.at[idx_b[o]], rows_b[o], sg[o])

                pltpu.make_async_copy(table.at[idx_b[b]], rows_b[b], sg[b]).wait()
                mul(b)

                @pl.when(g + 2 < NCH)
                def _():
                    pltpu.async_copy(srcg.at[tid, g + 2], idx_b[b], sf[b])
                    pltpu.async_copy(wexpg.at[tid, g + 2], wx_b[b], sf[b])

                pltpu.make_async_copy(dstg.at[tid, g], dst_b[b], sd[b]).wait()

                @pl.when(g < NCH - 1)
                def _():
                    pltpu.async_copy(rows_b[b], acc_sh.at[dst_b[b]], ss[b],
                                     add=True)

                @pl.when(g == NCH - 1)
                def _():
                    pltpu.sync_copy(rows_b[b], acc_sh.at[dst_b[b]], add=True)
            return 0

        lax.fori_loop(0, NCH // 2, kstep, 0)
        plsc.subcore_barrier()
        pltpu.sync_copy(acc_sh.at[pl.ds(s * RPS, RPS)],
                        out.at[c, pl.ds(s * RPS, RPS)])

    return pl.kernel(
        body,
        out_type=jax.ShapeDtypeStruct((2, NPAD, nf), _f32),
        mesh=mesh,
        compiler_params=pltpu.CompilerParams(needs_layout_passes=False,
                                             use_tc_tiling_on_sc=False),
        scratch_types=[
            pltpu.VMEM((CH,), jnp.int32),
            pltpu.VMEM((CH,), jnp.int32),
            pltpu.VMEM((CH,), jnp.int32),
            pltpu.VMEM((CH,), jnp.int32),
            pltpu.VMEM((CH,), _f32),
            pltpu.VMEM((CH,), _f32),
            pltpu.VMEM((CH, nf), _f32),
            pltpu.VMEM((CH, nf), _f32),
            pltpu.VMEM_SHARED((NPAD, nf), _f32),
            pltpu.SemaphoreType.DMA,
            pltpu.SemaphoreType.DMA,
            pltpu.SemaphoreType.DMA,
            pltpu.SemaphoreType.DMA,
            pltpu.SemaphoreType.DMA,
            pltpu.SemaphoreType.DMA,
            pltpu.SemaphoreType.DMA,
            pltpu.SemaphoreType.DMA,
        ],
    )


_spmm_x = _make_spmm(NFP, 2)
_spmm_cls = _make_spmm(NCLASS, 8)


def kernel(x, edge_index, edge_weight, W_org, b_org, Wg1, bg1, Wg2, bg2,
           W3, b3, Wl, bl):
    dst = edge_index[0].astype(jnp.int32).reshape(NTILES, NCH, CH)
    src = edge_index[1].astype(jnp.int32).reshape(NTILES, NCH, CH)
    w = edge_weight.reshape(NTILES, NCH, CH)

    onescol = jnp.broadcast_to(
        jnp.array([[1.0] + [0.0] * 15], _f32), (N, 16))
    xp = jnp.concatenate([x, onescol], axis=1)

    w1, c1, w2l, w3l, c3 = _fold(
        W_org, b_org.reshape(1, -1), Wg1, Wg2, Wl, W3,
        b3.reshape(1, -1), bg2.reshape(1, -1), bl.reshape(1, -1))

    z_x = jnp.zeros((RPS, NFP), _f32)
    p1 = _spmm_x(xp, src, dst, w, z_x)

    s2, r3 = _lin_b(p1, x, bg1.reshape(1, -1), w1, c1, w2l, w3l,
                    c3.reshape(1, -1))

    z_cls = jnp.zeros((RPS, NCLASS), _f32)
    p2 = _spmm_cls(s2, src, dst, w, z_cls)

    out = _final(r3.reshape(_NR, 128), p2[:, :N, :].reshape(2, _NR, 128))
    return out.reshape(N, NCLASS)


# speedup vs baseline: 10.1481x; 1.0000x over previous
import jax
import jax.numpy as jnp
from jax import lax
from jax.experimental import pallas as pl
from jax.experimental.pallas import tpu as pltpu
from jax.experimental.pallas import tpu_sc as plsc

N = 10000
E = 320000
NFEAT = 128
NHID = 64
NCLASS = 16
H2 = NHID * 2
H3 = NHID * 3
NFP = 144

NTILES = 32
EPT = E // NTILES
CH = 125
NCH = EPT // CH
NPAD = 10112
RPS = NPAD // 16

_f32 = jnp.float32


def _fold_body(w_org, b_org, wg1, wg2, wl, w3, b3, bg2, bl,
               w1_o, c1_o, w2l_o, w3l_o, c3_o):
    w_org_v = w_org[...]
    b_org_v = b_org[...]
    wl_v = wl[...]
    w3_v = w3[...]
    w1_o[...] = jnp.dot(w_org_v, wg1[...], preferred_element_type=_f32)
    c1_o[...] = jnp.dot(b_org_v, wg1[...], preferred_element_type=_f32)
    w2l_o[...] = jnp.dot(wg2[...], wl_v, preferred_element_type=_f32)
    w3l_o[...] = jnp.dot(jnp.dot(w_org_v, w3_v, preferred_element_type=_f32),
                         wl_v, preferred_element_type=_f32)
    c3_o[...] = (jnp.dot(jnp.dot(b_org_v, w3_v, preferred_element_type=_f32)
                         + b3[...] + 0.5 * bg2[...],
                         wl_v, preferred_element_type=_f32) + bl[...])


_fold = pl.pallas_call(
    _fold_body,
    out_shape=(
        jax.ShapeDtypeStruct((H2, H3), _f32),
        jax.ShapeDtypeStruct((1, H3), _f32),
        jax.ShapeDtypeStruct((H3, NCLASS), _f32),
        jax.ShapeDtypeStruct((H2, NCLASS), _f32),
        jax.ShapeDtypeStruct((1, NCLASS), _f32),
    ),
)


_BR = 1000


def _lin_b_body2(p1_hbm, x_ref, bg1_ref, w1_ref, c1_ref, w2l_ref, w3l_ref,
                 c3_ref, s2_hbm, r3_hbm, p1_v, s2_v, r3_v, sin, sout):
    i = pl.program_id(0)
    pltpu.make_async_copy(
        p1_hbm.at[:, pl.ds(i * _BR, _BR), :], p1_v, sin).start()
    pltpu.make_async_copy(
        p1_hbm.at[:, pl.ds(i * _BR, _BR), :], p1_v, sin).wait()
    pb = p1_v[...]
    ax = pb[0, :, :H2] + pb[1, :, :H2]
    deg = pb[0, :, H2:H2 + 1] + pb[1, :, H2:H2 + 1]
    a = (jnp.dot(ax, w1_ref[...], preferred_element_type=_f32)
         + deg * c1_ref[...] + bg1_ref[...])
    g1 = jnp.maximum(a, 0.0)
    s2_v[...] = jnp.dot(g1, w2l_ref[...], preferred_element_type=_f32)
    r3_v[...] = (jnp.dot(x_ref[...], w3l_ref[...],
                         preferred_element_type=_f32) + c3_ref[...])
    pltpu.make_async_copy(s2_v, s2_hbm.at[pl.ds(i * _BR, _BR)], sout).start()
    pltpu.make_async_copy(s2_v, s2_hbm.at[pl.ds(i * _BR, _BR)], sout).wait()
    pltpu.make_async_copy(r3_v, r3_hbm.at[pl.ds(i * _BR, _BR)], sout).start()
    pltpu.make_async_copy(r3_v, r3_hbm.at[pl.ds(i * _BR, _BR)], sout).wait()


_lin_b = pl.pallas_call(
    _lin_b_body2,
    grid=(N // _BR,),
    in_specs=[
        pl.BlockSpec(memory_space=pl.ANY),
        pl.BlockSpec((_BR, NFEAT), lambda i: (i, 0)),
        pl.BlockSpec((1, H3), lambda i: (0, 0)),
        pl.BlockSpec((H2, H3), lambda i: (0, 0)),
        pl.BlockSpec((1, H3), lambda i: (0, 0)),
        pl.BlockSpec((H3, NCLASS), lambda i: (0, 0)),
        pl.BlockSpec((H2, NCLASS), lambda i: (0, 0)),
        pl.BlockSpec((1, NCLASS), lambda i: (0, 0)),
    ],
    out_specs=(
        pl.BlockSpec(memory_space=pl.ANY),
        pl.BlockSpec(memory_space=pl.ANY),
    ),
    out_shape=(
        jax.ShapeDtypeStruct((N, NCLASS), _f32),
        jax.ShapeDtypeStruct((N, NCLASS), _f32),
    ),
    scratch_shapes=[
        pltpu.VMEM((2, _BR, NFP), _f32),
        pltpu.VMEM((_BR, NCLASS), _f32),
        pltpu.VMEM((_BR, NCLASS), _f32),
        pltpu.SemaphoreType.DMA,
        pltpu.SemaphoreType.DMA,
    ],
)


_NR = N * NCLASS // 128
_NRP = NPAD * NCLASS // 128


def _final_body(r3_ref, p2_ref, out_ref):
    p2 = p2_ref[...]
    t = r3_ref[...] + 0.5 * (p2[0, :_NR, :] + p2[1, :_NR, :])
    mn = jnp.min(t)
    mx = jnp.max(t)
    out_ref[...] = 2.0 * (t - mn) / (mx - mn) - 1.0


_final = pl.pallas_call(
    _final_body,
    out_shape=jax.ShapeDtypeStruct((_NR, 128), _f32),
)


def _make_spmm(nf, mul_unroll):
    nvec = nf // 16
    mesh = plsc.VectorSubcoreMesh(core_axis_name="c", subcore_axis_name="s")

    def body(table, srcg, dstg, wexpg, zrows, out,
             idx0, idx1, dst0, dst1, wx0, wx1, rows0, rows1, acc_sh,
             sf0, sf1, sd0, sd1, sg0, sg1, ss0, ss1):
        c = lax.axis_index("c")
        s = lax.axis_index("s")
        tid = s * 2 + c
        idx_b = (idx0, idx1)
        dst_b = (dst0, dst1)
        wx_b = (wx0, wx1)
        rows_b = (rows0, rows1)
        sf = (sf0, sf1)
        sd = (sd0, sd1)
        sg = (sg0, sg1)
        ss = (ss0, ss1)

        pltpu.sync_copy(zrows, acc_sh.at[pl.ds(s * RPS, RPS)])
        plsc.subcore_barrier()

        def mul(b):
            rows_v = rows_b[b]
            wx_v = wx_b[b]

            def row(r, _):
                wspl = plsc.load_gather(wx_v, [jnp.full((16,), r, jnp.int32)])
                for j in range(nvec):
                    rows_v[r, pl.ds(j * 16, 16)] = (
                        rows_v[r, pl.ds(j * 16, 16)] * wspl)
                return 0

            lax.fori_loop(0, CH, row, 0, unroll=mul_unroll)

        pltpu.async_copy(srcg.at[tid, 0], idx_b[0], sf[0])
        pltpu.async_copy(wexpg.at[tid, 0], wx_b[0], sf[0])
        pltpu.async_copy(srcg.at[tid, 1], idx_b[1], sf[1])
        pltpu.async_copy(wexpg.at[tid, 1], wx_b[1], sf[1])
        pltpu.async_copy(dstg.at[tid, 0], dst_b[0], sd[0])
        pltpu.make_async_copy(srcg.at[tid, 0], idx_b[0], sf[0]).wait()
        pltpu.make_async_copy(wexpg.at[tid, 0], wx_b[0], sf[0]).wait()
        pltpu.async_copy(table.at[idx_b[0]], rows_b[0], sg[0])

        def kstep(k, _):
            for b in (0, 1):
                g = 2 * k + b
                o = 1 - b

                @pl.when(g + 1 < NCH)
                def _():
                    pltpu.make_async_copy(
                        srcg.at[tid, g + 1], idx_b[o], sf[o]).wait()
                    pltpu.make_async_copy(
                        wexpg.at[tid, g + 1], wx_b[o], sf[o]).wait()

                @pl.when(g >= 1)
                def _():
                    pltpu.make_async_copy(
                        rows_b[o], acc_sh.at[dst_b[o]], ss[o]).wait()

                @pl.when(g + 1 < NCH)
                def _():
                    pltpu.async_copy(dstg.at[tid, g + 1], dst_b[o], sd[o])
                    pltpu.async_copy(table.at[idx_b[o]], rows_b[o], sg[o])

                pltpu.make_async_copy(table.at[idx_b[b]], rows_b[b], sg[b]).wait()
                mul(b)

                @pl.when(g + 2 < NCH)
                def _():
                    pltpu.async_copy(srcg.at[tid, g + 2], idx_b[b], sf[b])
                    pltpu.async_copy(wexpg.at[tid, g + 2], wx_b[b], sf[b])

                pltpu.make_async_copy(dstg.at[tid, g], dst_b[b], sd[b]).wait()

                @pl.when(g < NCH - 1)
                def _():
                    pltpu.async_copy(rows_b[b], acc_sh.at[dst_b[b]], ss[b],
                                     add=True)

                @pl.when(g == NCH - 1)
                def _():
                    pltpu.sync_copy(rows_b[b], acc_sh.at[dst_b[b]], add=True)
            return 0

        lax.fori_loop(0, NCH // 2, kstep, 0)
        plsc.subcore_barrier()
        pltpu.sync_copy(acc_sh.at[pl.ds(s * RPS, RPS)],
                        out.at[c, pl.ds(s * RPS, RPS)])

    return pl.kernel(
        body,
        out_type=jax.ShapeDtypeStruct((2, NPAD, nf), _f32),
        mesh=mesh,
        compiler_params=pltpu.CompilerParams(needs_layout_passes=False,
                                             use_tc_tiling_on_sc=False),
        scratch_types=[
            pltpu.VMEM((CH,), jnp.int32),
            pltpu.VMEM((CH,), jnp.int32),
            pltpu.VMEM((CH,), jnp.int32),
            pltpu.VMEM((CH,), jnp.int32),
            pltpu.VMEM((CH,), _f32),
            pltpu.VMEM((CH,), _f32),
            pltpu.VMEM((CH, nf), _f32),
            pltpu.VMEM((CH, nf), _f32),
            pltpu.VMEM_SHARED((NPAD, nf), _f32),
            pltpu.SemaphoreType.DMA,
            pltpu.SemaphoreType.DMA,
            pltpu.SemaphoreType.DMA,
            pltpu.SemaphoreType.DMA,
            pltpu.SemaphoreType.DMA,
            pltpu.SemaphoreType.DMA,
            pltpu.SemaphoreType.DMA,
            pltpu.SemaphoreType.DMA,
        ],
    )


_spmm_x = _make_spmm(NFP, 2)
_spmm_cls = _make_spmm(NCLASS, 8)


def kernel(x, edge_index, edge_weight, W_org, b_org, Wg1, bg1, Wg2, bg2,
           W3, b3, Wl, bl):
    dst = edge_index[0].astype(jnp.int32).reshape(NTILES, NCH, CH)
    src = edge_index[1].astype(jnp.int32).reshape(NTILES, NCH, CH)
    w = edge_weight.reshape(NTILES, NCH, CH)

    onescol = jnp.broadcast_to(
        jnp.array([[1.0] + [0.0] * 15], _f32), (N, 16))
    xp = jnp.concatenate([x, onescol], axis=1)

    w1, c1, w2l, w3l, c3 = _fold(
        W_org, b_org.reshape(1, -1), Wg1, Wg2, Wl, W3,
        b3.reshape(1, -1), bg2.reshape(1, -1), bl.reshape(1, -1))

    z_x = jnp.zeros((RPS, NFP), _f32)
    p1 = _spmm_x(xp, src, dst, w, z_x)

    s2, r3 = _lin_b(p1, x, bg1.reshape(1, -1), w1, c1, w2l, w3l,
                    c3.reshape(1, -1))

    z_cls = jnp.zeros((RPS, NCLASS), _f32)
    p2 = _spmm_cls(s2, src, dst, w, z_cls)

    out = _final(r3.reshape(_NR, 128), p2.reshape(2, _NRP, 128))
    return out.reshape(N, NCLASS)
